# baseline (speedup 1.0000x reference)
import functools

import jax
import jax.numpy as jnp
from jax.experimental import pallas as pl
from jax.experimental.pallas import tpu as pltpu

N = 32
N_PRIORS = 8732
N_CLASSES = 81
N_OBJ = 12
THRESHOLD = 0.5
NEG_POS_RATIO = 3
ALPHA = 1.0


def _image_kernel(offsets_ref, scores_ref, boxes_ref, labels_ref, priors_ref,
                  out_ref):
    f32 = jnp.float32
    i32 = jnp.int32
    lane = jax.lax.broadcasted_iota(i32, (1, N_PRIORS), 1)

    pcx = priors_ref[0:1, :]
    pcy = priors_ref[1:2, :]
    pw = priors_ref[2:3, :]
    ph = priors_ref[3:4, :]
    px1 = pcx - pw * 0.5
    py1 = pcy - ph * 0.5
    px2 = pcx + pw * 0.5
    py2 = pcy + ph * 0.5
    parea = pw * ph

    best_iou = jnp.zeros((1, N_PRIORS), f32)
    best_obj = jnp.zeros((1, N_PRIORS), i32)
    forced_obj = jnp.zeros((1, N_PRIORS), i32)
    forced_any = jnp.zeros((1, N_PRIORS), jnp.bool_)

    for j in range(N_OBJ):
        bx1 = boxes_ref[0, j, 0]
        by1 = boxes_ref[0, j, 1]
        bx2 = boxes_ref[0, j, 2]
        by2 = boxes_ref[0, j, 3]
        barea = (bx2 - bx1) * (by2 - by1)
        iw = jnp.minimum(px2, bx2) - jnp.maximum(px1, bx1)
        ih = jnp.minimum(py2, by2) - jnp.maximum(py1, by1)
        iw = jnp.maximum(iw, 0.0)
        ih = jnp.maximum(ih, 0.0)
        inter = iw * ih
        iou = inter / (parea + barea - inter)
        upd = iou > best_iou
        best_iou = jnp.where(upd, iou, best_iou)
        best_obj = jnp.where(upd, j, best_obj)
        mj = jnp.max(iou)
        oj = jnp.min(jnp.where(iou == mj, lane, N_PRIORS))
        hit = lane == oj
        forced_obj = jnp.where(hit, j, forced_obj)
        forced_any = jnp.logical_or(forced_any, hit)

    obj_eff = jnp.where(forced_any, forced_obj, best_obj)
    iou_eff = jnp.where(forced_any, 1.0, best_iou)

    cls_row = jnp.zeros((1, N_PRIORS), i32)
    bcx = jnp.zeros((1, N_PRIORS), f32)
    bcy = jnp.zeros((1, N_PRIORS), f32)
    lbw = jnp.zeros((1, N_PRIORS), f32)
    lbh = jnp.zeros((1, N_PRIORS), f32)
    for j in range(N_OBJ):
        sel = obj_eff == j
        cls_row = jnp.where(sel, labels_ref[0, 0, j], cls_row)
        bx1 = boxes_ref[0, j, 0]
        by1 = boxes_ref[0, j, 1]
        bx2 = boxes_ref[0, j, 2]
        by2 = boxes_ref[0, j, 3]
        bcx = jnp.where(sel, (bx1 + bx2) * 0.5, bcx)
        bcy = jnp.where(sel, (by1 + by2) * 0.5, bcy)
        lbw = jnp.where(sel, jnp.log(bx2 - bx1), lbw)
        lbh = jnp.where(sel, jnp.log(by2 - by1), lbh)

    cls_row = jnp.where(iou_eff < THRESHOLD, 0, cls_row)
    pos = cls_row != 0
    pos_f = pos.astype(f32)
    n_pos = jnp.sum(pos_f)

    t0 = (bcx - pcx) * 10.0 / pw
    t1 = (bcy - pcy) * 10.0 / ph
    t2 = (lbw - jnp.log(pw)) * 5.0
    t3 = (lbh - jnp.log(ph)) * 5.0
    loc_sum = jnp.zeros((), f32)
    for k, t in enumerate((t0, t1, t2, t3)):
        d = offsets_ref[0, k:k + 1, :] - t
        ad = jnp.abs(d)
        sl1 = jnp.where(ad < 1.0, 0.5 * d * d, ad - 0.5)
        loc_sum = loc_sum + jnp.sum(sl1 * pos_f)

    s = scores_ref[0]
    m = jnp.max(s, axis=1, keepdims=True)
    e = jnp.exp(s - m)
    lse = m + jnp.log(jnp.sum(e, axis=1, keepdims=True))
    cls_col = cls_row.reshape(N_PRIORS, 1)
    cio = jax.lax.broadcasted_iota(i32, (N_PRIORS, N_CLASSES), 1)
    selsc = jnp.sum(jnp.where(cio == cls_col, s, 0.0), axis=1, keepdims=True)
    ce_col = lse - selsc
    ce_row = ce_col.reshape(1, N_PRIORS)

    pos_ce = jnp.sum(ce_row * pos_f)

    neg = jnp.where(pos, 0.0, ce_row)
    nbits = jax.lax.bitcast_convert_type(neg, i32)
    k_hard = (NEG_POS_RATIO * n_pos).astype(i32)

    def body(_, carry):
        lo, hi = carry
        mid = lo + (hi - lo + 1) // 2
        cnt = jnp.sum((nbits >= mid).astype(i32))
        ok = cnt >= k_hard
        return jnp.where(ok, mid, lo), jnp.where(ok, hi, mid - 1)

    lo, _ = jax.lax.fori_loop(
        0, 31, body,
        (jnp.zeros((), i32), jnp.asarray(0x7F7FFFFF, i32)))
    kth = jnp.max(jnp.where(nbits == lo, neg, 0.0))
    gt = neg > kth
    cnt_gt = jnp.sum(gt.astype(i32))
    hard = jnp.sum(jnp.where(gt, neg, 0.0)) + (
        (k_hard - cnt_gt).astype(f32) * kth)

    out_ref[0, 0, 0] = loc_sum
    out_ref[0, 0, 1] = n_pos
    out_ref[0, 0, 2] = pos_ce
    out_ref[0, 0, 3] = hard


@jax.jit
def kernel(predicted_offsets, predicted_scores, boxes, labels, priors_cxcy):
    offsets_t = jnp.transpose(predicted_offsets, (0, 2, 1))
    priors_t = priors_cxcy.T
    labels = labels.astype(jnp.int32).reshape(N, 1, N_OBJ)

    parts = pl.pallas_call(
        _image_kernel,
        grid=(N,),
        in_specs=[
            pl.BlockSpec((1, 4, N_PRIORS), lambda i: (i, 0, 0)),
            pl.BlockSpec((1, N_PRIORS, N_CLASSES), lambda i: (i, 0, 0)),
            pl.BlockSpec((1, N_OBJ, 4), lambda i: (i, 0, 0),
                         memory_space=pltpu.SMEM),
            pl.BlockSpec((1, 1, N_OBJ), lambda i: (i, 0, 0),
                         memory_space=pltpu.SMEM),
            pl.BlockSpec((4, N_PRIORS), lambda i: (0, 0)),
        ],
        out_specs=pl.BlockSpec((1, 1, 4), lambda i: (i, 0, 0),
                               memory_space=pltpu.SMEM),
        out_shape=jax.ShapeDtypeStruct((N, 1, 4), jnp.float32),
        compiler_params=pltpu.CompilerParams(
            dimension_semantics=("parallel",)),
    )(offsets_t, predicted_scores, boxes, labels, priors_t)

    loc_sum = parts[:, 0, 0].sum()
    n_pos = parts[:, 0, 1].sum()
    conf = parts[:, 0, 2].sum() + parts[:, 0, 3].sum()
    return ALPHA * loc_sum / (n_pos * 4.0) + conf / n_pos


# speedup vs baseline: 1.2452x; 1.2452x over previous
import jax
import jax.numpy as jnp
from jax.experimental import pallas as pl
from jax.experimental.pallas import tpu as pltpu

N = 32
N_PRIORS = 8732
N_CLASSES = 81
N_OBJ = 12
THRESHOLD = 0.5
NEG_POS_RATIO = 3
ALPHA = 1.0


def _match_ce_kernel(offsets_ref, scores_ref, boxes_ref, labels_ref,
                     priors_ref, neg_ref, parts_ref):
    f32 = jnp.float32
    i32 = jnp.int32
    lane = jax.lax.broadcasted_iota(i32, (1, N_PRIORS), 1)

    pcx = priors_ref[0:1, :]
    pcy = priors_ref[1:2, :]
    pw = priors_ref[2:3, :]
    ph = priors_ref[3:4, :]
    px1 = pcx - pw * 0.5
    py1 = pcy - ph * 0.5
    px2 = pcx + pw * 0.5
    py2 = pcy + ph * 0.5
    parea = pw * ph

    best_iou = jnp.zeros((1, N_PRIORS), f32)
    best_obj = jnp.zeros((1, N_PRIORS), i32)
    forced_obj = jnp.zeros((1, N_PRIORS), i32)
    forced_any = jnp.zeros((1, N_PRIORS), jnp.bool_)

    for j in range(N_OBJ):
        bx1 = boxes_ref[0, j, 0]
        by1 = boxes_ref[0, j, 1]
        bx2 = boxes_ref[0, j, 2]
        by2 = boxes_ref[0, j, 3]
        barea = (bx2 - bx1) * (by2 - by1)
        iw = jnp.minimum(px2, bx2) - jnp.maximum(px1, bx1)
        ih = jnp.minimum(py2, by2) - jnp.maximum(py1, by1)
        iw = jnp.maximum(iw, 0.0)
        ih = jnp.maximum(ih, 0.0)
        inter = iw * ih
        iou = inter / (parea + barea - inter)
        upd = iou > best_iou
        best_iou = jnp.where(upd, iou, best_iou)
        best_obj = jnp.where(upd, j, best_obj)
        mj = jnp.max(iou)
        oj = jnp.min(jnp.where(iou == mj, lane, N_PRIORS))
        hit = lane == oj
        forced_obj = jnp.where(hit, j, forced_obj)
        forced_any = jnp.logical_or(forced_any, hit)

    obj_eff = jnp.where(forced_any, forced_obj, best_obj)
    iou_eff = jnp.where(forced_any, 1.0, best_iou)

    cls_row = jnp.zeros((1, N_PRIORS), i32)
    bcx = jnp.zeros((1, N_PRIORS), f32)
    bcy = jnp.zeros((1, N_PRIORS), f32)
    lbw = jnp.zeros((1, N_PRIORS), f32)
    lbh = jnp.zeros((1, N_PRIORS), f32)
    for j in range(N_OBJ):
        sel = obj_eff == j
        cls_row = jnp.where(sel, labels_ref[0, 0, j], cls_row)
        bx1 = boxes_ref[0, j, 0]
        by1 = boxes_ref[0, j, 1]
        bx2 = boxes_ref[0, j, 2]
        by2 = boxes_ref[0, j, 3]
        bcx = jnp.where(sel, (bx1 + bx2) * 0.5, bcx)
        bcy = jnp.where(sel, (by1 + by2) * 0.5, bcy)
        lbw = jnp.where(sel, jnp.log(bx2 - bx1), lbw)
        lbh = jnp.where(sel, jnp.log(by2 - by1), lbh)

    cls_row = jnp.where(iou_eff < THRESHOLD, 0, cls_row)
    pos = cls_row != 0
    pos_f = pos.astype(f32)
    n_pos = jnp.sum(pos_f)

    t0 = (bcx - pcx) * 10.0 / pw
    t1 = (bcy - pcy) * 10.0 / ph
    t2 = (lbw - jnp.log(pw)) * 5.0
    t3 = (lbh - jnp.log(ph)) * 5.0
    loc_sum = jnp.zeros((), f32)
    for k, t in enumerate((t0, t1, t2, t3)):
        d = offsets_ref[0, k:k + 1, :] - t
        ad = jnp.abs(d)
        sl1 = jnp.where(ad < 1.0, 0.5 * d * d, ad - 0.5)
        loc_sum = loc_sum + jnp.sum(sl1 * pos_f)

    s = scores_ref[0]
    m = jnp.max(s, axis=1, keepdims=True)
    e = jnp.exp(s - m)
    lse = m + jnp.log(jnp.sum(e, axis=1, keepdims=True))
    cls_col = cls_row.reshape(N_PRIORS, 1)
    cio = jax.lax.broadcasted_iota(i32, (N_PRIORS, N_CLASSES), 1)
    selsc = jnp.sum(jnp.where(cio == cls_col, s, 0.0), axis=1, keepdims=True)
    ce_col = lse - selsc
    ce_row = ce_col.reshape(1, N_PRIORS)

    pos_ce = jnp.sum(ce_row * pos_f)
    neg_ref[0, 0, :] = jnp.where(pos, 0.0, ce_row)[0]

    parts_ref[0, 0, 0] = loc_sum
    parts_ref[0, 0, 1] = n_pos
    parts_ref[0, 0, 2] = pos_ce


def _mine_kernel(neg_ref, parts_ref, out_ref):
    f32 = jnp.float32
    i32 = jnp.int32
    neg = neg_ref[:, 0, :]
    nbits = jax.lax.bitcast_convert_type(neg, i32)
    n_pos_col = parts_ref[:, 0, 1:2]
    k_hard = (NEG_POS_RATIO * n_pos_col).astype(i32)

    def body(_, carry):
        lo, hi = carry
        mid = lo + (hi - lo + 1) // 2
        cnt = jnp.sum((nbits >= mid).astype(i32), axis=1, keepdims=True)
        ok = cnt >= k_hard
        return jnp.where(ok, mid, lo), jnp.where(ok, hi, mid - 1)

    lo, _ = jax.lax.fori_loop(
        0, 31, body,
        (jnp.zeros((N, 1), i32), jnp.full((N, 1), 0x7F7FFFFF, i32)))
    kth = jnp.max(jnp.where(nbits == lo, neg, 0.0), axis=1, keepdims=True)
    gt = neg > kth
    cnt_gt = jnp.sum(gt.astype(i32), axis=1, keepdims=True)
    hard = jnp.sum(jnp.where(gt, neg, 0.0), axis=1, keepdims=True) + (
        (k_hard - cnt_gt).astype(f32) * kth)

    loc_sum = jnp.sum(parts_ref[:, 0, 0:1])
    n_pos = jnp.sum(parts_ref[:, 0, 1:2])
    conf = jnp.sum(parts_ref[:, 0, 2:3]) + jnp.sum(hard)
    out_ref[0, 0] = ALPHA * loc_sum / (n_pos * 4.0) + conf / n_pos


@jax.jit
def kernel(predicted_offsets, predicted_scores, boxes, labels, priors_cxcy):
    offsets_t = jnp.transpose(predicted_offsets, (0, 2, 1))
    priors_t = priors_cxcy.T
    labels = labels.astype(jnp.int32).reshape(N, 1, N_OBJ)

    neg, parts = pl.pallas_call(
        _match_ce_kernel,
        grid=(N,),
        in_specs=[
            pl.BlockSpec((1, 4, N_PRIORS), lambda i: (i, 0, 0)),
            pl.BlockSpec((1, N_PRIORS, N_CLASSES), lambda i: (i, 0, 0)),
            pl.BlockSpec((1, N_OBJ, 4), lambda i: (i, 0, 0),
                         memory_space=pltpu.SMEM),
            pl.BlockSpec((1, 1, N_OBJ), lambda i: (i, 0, 0),
                         memory_space=pltpu.SMEM),
            pl.BlockSpec((4, N_PRIORS), lambda i: (0, 0)),
        ],
        out_specs=[
            pl.BlockSpec((1, 1, N_PRIORS), lambda i: (i, 0, 0)),
            pl.BlockSpec((1, 1, 4), lambda i: (i, 0, 0),
                         memory_space=pltpu.SMEM),
        ],
        out_shape=[
            jax.ShapeDtypeStruct((N, 1, N_PRIORS), jnp.float32),
            jax.ShapeDtypeStruct((N, 1, 4), jnp.float32),
        ],
        compiler_params=pltpu.CompilerParams(
            dimension_semantics=("parallel",)),
    )(offsets_t, predicted_scores, boxes, labels, priors_t)

    loss = pl.pallas_call(
        _mine_kernel,
        in_specs=[
            pl.BlockSpec((N, 1, N_PRIORS), lambda: (0, 0, 0)),
            pl.BlockSpec((N, 1, 4), lambda: (0, 0, 0)),
        ],
        out_specs=pl.BlockSpec((1, 1), lambda: (0, 0),
                               memory_space=pltpu.SMEM),
        out_shape=jax.ShapeDtypeStruct((1, 1), jnp.float32),
    )(neg, parts)

    return loss[0, 0]


# speedup vs baseline: 1.4351x; 1.1524x over previous
import jax
import jax.numpy as jnp
from jax.experimental import pallas as pl
from jax.experimental.pallas import tpu as pltpu

N = 32
N_PRIORS = 8732
N_CLASSES = 81
N_OBJ = 12
THRESHOLD = 0.5
NEG_POS_RATIO = 3
ALPHA = 1.0


def _match_kernel(offsets_ref, boxes_ref, labels_ref, priors_ref,
                  cls_ref, loc_ref, npos_ref):
    f32 = jnp.float32
    i32 = jnp.int32
    lane = jax.lax.broadcasted_iota(i32, (1, N_PRIORS), 1)

    pcx = priors_ref[0:1, :]
    pcy = priors_ref[1:2, :]
    pw = priors_ref[2:3, :]
    ph = priors_ref[3:4, :]
    px1 = pcx - pw * 0.5
    py1 = pcy - ph * 0.5
    px2 = pcx + pw * 0.5
    py2 = pcy + ph * 0.5
    parea = pw * ph

    best_iou = jnp.zeros((N, N_PRIORS), f32)
    best_obj = jnp.zeros((N, N_PRIORS), i32)
    forced_obj = jnp.zeros((N, N_PRIORS), i32)
    forced_any = jnp.zeros((N, N_PRIORS), jnp.bool_)

    for j in range(N_OBJ):
        bx1 = boxes_ref[j, 0]
        by1 = boxes_ref[j, 1]
        bx2 = boxes_ref[j, 2]
        by2 = boxes_ref[j, 3]
        barea = (bx2 - bx1) * (by2 - by1)
        iw = jnp.minimum(px2, bx2) - jnp.maximum(px1, bx1)
        ih = jnp.minimum(py2, by2) - jnp.maximum(py1, by1)
        iw = jnp.maximum(iw, 0.0)
        ih = jnp.maximum(ih, 0.0)
        inter = iw * ih
        iou = inter / (parea + barea - inter)
        upd = iou > best_iou
        best_iou = jnp.where(upd, iou, best_iou)
        best_obj = jnp.where(upd, j, best_obj)
        mj = jnp.max(iou, axis=1, keepdims=True)
        oj = jnp.min(jnp.where(iou == mj, lane, N_PRIORS), axis=1,
                     keepdims=True)
        hit = lane == oj
        forced_obj = jnp.where(hit, j, forced_obj)
        forced_any = jnp.logical_or(forced_any, hit)

    obj_eff = jnp.where(forced_any, forced_obj, best_obj)
    iou_eff = jnp.where(forced_any, 1.0, best_iou)

    cls = jnp.zeros((N, N_PRIORS), i32)
    bcx = jnp.zeros((N, N_PRIORS), f32)
    bcy = jnp.zeros((N, N_PRIORS), f32)
    lbw = jnp.zeros((N, N_PRIORS), f32)
    lbh = jnp.zeros((N, N_PRIORS), f32)
    for j in range(N_OBJ):
        sel = obj_eff == j
        cls = jnp.where(sel, labels_ref[j], cls)
        bx1 = boxes_ref[j, 0]
        by1 = boxes_ref[j, 1]
        bx2 = boxes_ref[j, 2]
        by2 = boxes_ref[j, 3]
        bcx = jnp.where(sel, (bx1 + bx2) * 0.5, bcx)
        bcy = jnp.where(sel, (by1 + by2) * 0.5, bcy)
        lbw = jnp.where(sel, jnp.log(bx2 - bx1), lbw)
        lbh = jnp.where(sel, jnp.log(by2 - by1), lbh)

    cls = jnp.where(iou_eff < THRESHOLD, 0, cls)
    pos_f = (cls != 0).astype(f32)
    npos_ref[:, :] = jnp.sum(pos_f, axis=1, keepdims=True)

    t0 = (bcx - pcx) * 10.0 / pw
    t1 = (bcy - pcy) * 10.0 / ph
    t2 = (lbw - jnp.log(pw)) * 5.0
    t3 = (lbh - jnp.log(ph)) * 5.0
    loc = jnp.zeros((N, 1), f32)
    for k, t in enumerate((t0, t1, t2, t3)):
        d = offsets_ref[:, k, :] - t
        ad = jnp.abs(d)
        sl1 = jnp.where(ad < 1.0, 0.5 * d * d, ad - 0.5)
        loc = loc + jnp.sum(sl1 * pos_f, axis=1, keepdims=True)
    loc_ref[:, :] = loc
    cls_ref[:, 0, :] = cls


def _ce_kernel(scores_ref, cls_ref, neg_ref, posce_ref):
    i32 = jnp.int32
    s = scores_ref[0]
    e = jnp.exp(s)
    lse = jnp.log(jnp.sum(e, axis=1, keepdims=True))
    cls_col = cls_ref[0, 0, :].reshape(N_PRIORS, 1)
    cio = jax.lax.broadcasted_iota(i32, (N_PRIORS, N_CLASSES), 1)
    selsc = jnp.sum(jnp.where(cio == cls_col, s, 0.0), axis=1, keepdims=True)
    ce_row = (lse - selsc).reshape(1, N_PRIORS)
    pos = cls_ref[0, 0, :].reshape(1, N_PRIORS) != 0
    posce_ref[0, 0, 0] = jnp.sum(jnp.where(pos, ce_row, 0.0))
    neg_ref[0, 0, :] = jnp.where(pos, 0.0, ce_row)[0]


def _mine_kernel(neg_ref, posce_ref, loc_partial_ref, npos_ref, out_ref):
    f32 = jnp.float32
    i32 = jnp.int32
    neg = neg_ref[:, 0, :]
    nbits = jax.lax.bitcast_convert_type(neg, i32)
    k_hard = (NEG_POS_RATIO * npos_ref[:, :]).astype(i32)

    def body(_, carry):
        lo, hi = carry
        mid = lo + (hi - lo + 1) // 2
        cnt = jnp.sum((nbits >= mid).astype(i32), axis=1, keepdims=True)
        ok = cnt >= k_hard
        return jnp.where(ok, mid, lo), jnp.where(ok, hi, mid - 1)

    lo, _ = jax.lax.fori_loop(
        0, 31, body,
        (jnp.zeros((N, 1), i32), jnp.full((N, 1), 0x7F7FFFFF, i32)))
    kth = jnp.max(jnp.where(nbits == lo, neg, 0.0), axis=1, keepdims=True)
    gt = neg > kth
    cnt_gt = jnp.sum(gt.astype(i32), axis=1, keepdims=True)
    hard = jnp.sum(jnp.where(gt, neg, 0.0), axis=1, keepdims=True) + (
        (k_hard - cnt_gt).astype(f32) * kth)

    loc_sum = jnp.sum(loc_partial_ref[:, :])
    n_pos = jnp.sum(npos_ref[:, :])
    conf = jnp.sum(posce_ref[:, 0, :]) + jnp.sum(hard)
    out_ref[0, 0] = ALPHA * loc_sum / (n_pos * 4.0) + conf / n_pos


@jax.jit
def kernel(predicted_offsets, predicted_scores, boxes, labels, priors_cxcy):
    offsets_t = jnp.transpose(predicted_offsets, (0, 2, 1))
    priors_t = priors_cxcy.T
    boxes_r = jnp.transpose(boxes, (1, 2, 0))[:, :, :, None]
    labels_r = labels.astype(jnp.int32).T[:, :, None]

    cls, loc_p, npos = pl.pallas_call(
        _match_kernel,
        in_specs=[
            pl.BlockSpec((N, 4, N_PRIORS), lambda: (0, 0, 0)),
            pl.BlockSpec((N_OBJ, 4, N, 1), lambda: (0, 0, 0, 0)),
            pl.BlockSpec((N_OBJ, N, 1), lambda: (0, 0, 0)),
            pl.BlockSpec((4, N_PRIORS), lambda: (0, 0)),
        ],
        out_specs=[
            pl.BlockSpec((N, 1, N_PRIORS), lambda: (0, 0, 0)),
            pl.BlockSpec((N, 1), lambda: (0, 0)),
            pl.BlockSpec((N, 1), lambda: (0, 0)),
        ],
        out_shape=[
            jax.ShapeDtypeStruct((N, 1, N_PRIORS), jnp.int32),
            jax.ShapeDtypeStruct((N, 1), jnp.float32),
            jax.ShapeDtypeStruct((N, 1), jnp.float32),
        ],
    )(offsets_t, boxes_r, labels_r, priors_t)

    neg, posce = pl.pallas_call(
        _ce_kernel,
        grid=(N,),
        in_specs=[
            pl.BlockSpec((1, N_PRIORS, N_CLASSES), lambda i: (i, 0, 0)),
            pl.BlockSpec((1, 1, N_PRIORS), lambda i: (i, 0, 0)),
        ],
        out_specs=[
            pl.BlockSpec((1, 1, N_PRIORS), lambda i: (i, 0, 0)),
            pl.BlockSpec((1, 1, 1), lambda i: (i, 0, 0),
                         memory_space=pltpu.SMEM),
        ],
        out_shape=[
            jax.ShapeDtypeStruct((N, 1, N_PRIORS), jnp.float32),
            jax.ShapeDtypeStruct((N, 1, 1), jnp.float32),
        ],
        compiler_params=pltpu.CompilerParams(
            dimension_semantics=("arbitrary",)),
    )(predicted_scores, cls)

    loss = pl.pallas_call(
        _mine_kernel,
        in_specs=[
            pl.BlockSpec((N, 1, N_PRIORS), lambda: (0, 0, 0)),
            pl.BlockSpec((N, 1, 1), lambda: (0, 0, 0)),
            pl.BlockSpec((N, 1), lambda: (0, 0)),
            pl.BlockSpec((N, 1), lambda: (0, 0)),
        ],
        out_specs=pl.BlockSpec((1, 1), lambda: (0, 0),
                               memory_space=pltpu.SMEM),
        out_shape=jax.ShapeDtypeStruct((1, 1), jnp.float32),
    )(neg, posce, loc_p, npos)

    return loss[0, 0]


# speedup vs baseline: 2.4736x; 1.7237x over previous
import jax
import jax.numpy as jnp
from jax.experimental import pallas as pl
from jax.experimental.pallas import tpu as pltpu

N = 32
N_PRIORS = 8732
N_CLASSES = 81
N_OBJ = 12
THRESHOLD = 0.5
NEG_POS_RATIO = 3
ALPHA = 1.0


def _match_kernel(offsets_ref, boxes_ref, labels_ref, priors_ref,
                  cls_ref, loc_ref, npos_ref):
    f32 = jnp.float32
    i32 = jnp.int32
    lane = jax.lax.broadcasted_iota(i32, (1, N_PRIORS), 1)

    pcx = priors_ref[0:1, :]
    pcy = priors_ref[1:2, :]
    pw = priors_ref[2:3, :]
    ph = priors_ref[3:4, :]
    px1 = pcx - pw * 0.5
    py1 = pcy - ph * 0.5
    px2 = pcx + pw * 0.5
    py2 = pcy + ph * 0.5
    parea = pw * ph

    best_iou = jnp.zeros((N, N_PRIORS), f32)
    best_obj = jnp.zeros((N, N_PRIORS), i32)
    forced_obj = jnp.zeros((N, N_PRIORS), i32)
    forced_any = jnp.zeros((N, N_PRIORS), jnp.bool_)

    for j in range(N_OBJ):
        bx1 = boxes_ref[j, 0]
        by1 = boxes_ref[j, 1]
        bx2 = boxes_ref[j, 2]
        by2 = boxes_ref[j, 3]
        barea = (bx2 - bx1) * (by2 - by1)
        iw = jnp.minimum(px2, bx2) - jnp.maximum(px1, bx1)
        ih = jnp.minimum(py2, by2) - jnp.maximum(py1, by1)
        iw = jnp.maximum(iw, 0.0)
        ih = jnp.maximum(ih, 0.0)
        inter = iw * ih
        iou = inter / (parea + barea - inter)
        upd = iou > best_iou
        best_iou = jnp.where(upd, iou, best_iou)
        best_obj = jnp.where(upd, j, best_obj)
        mj = jnp.max(iou, axis=1, keepdims=True)
        oj = jnp.min(jnp.where(iou == mj, lane, N_PRIORS), axis=1,
                     keepdims=True)
        hit = lane == oj
        forced_obj = jnp.where(hit, j, forced_obj)
        forced_any = jnp.logical_or(forced_any, hit)

    obj_eff = jnp.where(forced_any, forced_obj, best_obj)
    iou_eff = jnp.where(forced_any, 1.0, best_iou)

    cls = jnp.zeros((N, N_PRIORS), i32)
    bcx = jnp.zeros((N, N_PRIORS), f32)
    bcy = jnp.zeros((N, N_PRIORS), f32)
    lbw = jnp.zeros((N, N_PRIORS), f32)
    lbh = jnp.zeros((N, N_PRIORS), f32)
    for j in range(N_OBJ):
        sel = obj_eff == j
        cls = jnp.where(sel, labels_ref[j], cls)
        bx1 = boxes_ref[j, 0]
        by1 = boxes_ref[j, 1]
        bx2 = boxes_ref[j, 2]
        by2 = boxes_ref[j, 3]
        bcx = jnp.where(sel, (bx1 + bx2) * 0.5, bcx)
        bcy = jnp.where(sel, (by1 + by2) * 0.5, bcy)
        lbw = jnp.where(sel, jnp.log(bx2 - bx1), lbw)
        lbh = jnp.where(sel, jnp.log(by2 - by1), lbh)

    cls = jnp.where(iou_eff < THRESHOLD, 0, cls)
    pos_f = (cls != 0).astype(f32)
    npos_ref[:, :] = jnp.sum(pos_f, axis=1, keepdims=True)

    t0 = (bcx - pcx) * 10.0 / pw
    t1 = (bcy - pcy) * 10.0 / ph
    t2 = (lbw - jnp.log(pw)) * 5.0
    t3 = (lbh - jnp.log(ph)) * 5.0
    loc = jnp.zeros((N, 1), f32)
    for k, t in enumerate((t0, t1, t2, t3)):
        d = offsets_ref[:, k, :] - t
        ad = jnp.abs(d)
        sl1 = jnp.where(ad < 1.0, 0.5 * d * d, ad - 0.5)
        loc = loc + jnp.sum(sl1 * pos_f, axis=1, keepdims=True)
    loc_ref[:, :] = loc
    cls_ref[:, 0, :] = cls


def _ce_kernel(scores_ref, cls_ref, neg_ref, posce_ref):
    i32 = jnp.int32
    st = scores_ref[0].T
    e = jnp.exp(st)
    lse = jnp.log(jnp.sum(e, axis=0, keepdims=True))
    cls_row = cls_ref[0, 0, :].reshape(1, N_PRIORS)
    rio = jax.lax.broadcasted_iota(i32, (N_CLASSES, N_PRIORS), 0)
    selsc = jnp.sum(jnp.where(rio == cls_row, st, 0.0), axis=0, keepdims=True)
    ce_row = lse - selsc
    pos = cls_row != 0
    posce_ref[0, 0, 0] = jnp.sum(jnp.where(pos, ce_row, 0.0))
    neg_ref[0, 0, :] = jnp.where(pos, 0.0, ce_row)[0]


def _mine_kernel(neg_ref, posce_ref, loc_partial_ref, npos_ref, out_ref):
    f32 = jnp.float32
    i32 = jnp.int32
    neg = neg_ref[:, 0, :]
    nbits = jax.lax.bitcast_convert_type(neg, i32)
    k_hard = (NEG_POS_RATIO * npos_ref[:, :]).astype(i32)

    def body(_, carry):
        lo, hi = carry
        mid = lo + (hi - lo + 1) // 2
        cnt = jnp.sum((nbits >= mid).astype(i32), axis=1, keepdims=True)
        ok = cnt >= k_hard
        return jnp.where(ok, mid, lo), jnp.where(ok, hi, mid - 1)

    lo, _ = jax.lax.fori_loop(
        0, 31, body,
        (jnp.zeros((N, 1), i32), jnp.full((N, 1), 0x7F7FFFFF, i32)))
    kth = jnp.max(jnp.where(nbits == lo, neg, 0.0), axis=1, keepdims=True)
    gt = neg > kth
    cnt_gt = jnp.sum(gt.astype(i32), axis=1, keepdims=True)
    hard = jnp.sum(jnp.where(gt, neg, 0.0), axis=1, keepdims=True) + (
        (k_hard - cnt_gt).astype(f32) * kth)

    loc_sum = jnp.sum(loc_partial_ref[:, :])
    n_pos = jnp.sum(npos_ref[:, :])
    conf = jnp.sum(posce_ref[:, 0, :]) + jnp.sum(hard)
    out_ref[0, 0] = ALPHA * loc_sum / (n_pos * 4.0) + conf / n_pos


@jax.jit
def kernel(predicted_offsets, predicted_scores, boxes, labels, priors_cxcy):
    offsets_t = jnp.transpose(predicted_offsets, (0, 2, 1))
    priors_t = priors_cxcy.T
    boxes_r = jnp.transpose(boxes, (1, 2, 0))[:, :, :, None]
    labels_r = labels.astype(jnp.int32).T[:, :, None]

    cls, loc_p, npos = pl.pallas_call(
        _match_kernel,
        in_specs=[
            pl.BlockSpec((N, 4, N_PRIORS), lambda: (0, 0, 0)),
            pl.BlockSpec((N_OBJ, 4, N, 1), lambda: (0, 0, 0, 0)),
            pl.BlockSpec((N_OBJ, N, 1), lambda: (0, 0, 0)),
            pl.BlockSpec((4, N_PRIORS), lambda: (0, 0)),
        ],
        out_specs=[
            pl.BlockSpec((N, 1, N_PRIORS), lambda: (0, 0, 0)),
            pl.BlockSpec((N, 1), lambda: (0, 0)),
            pl.BlockSpec((N, 1), lambda: (0, 0)),
        ],
        out_shape=[
            jax.ShapeDtypeStruct((N, 1, N_PRIORS), jnp.int32),
            jax.ShapeDtypeStruct((N, 1), jnp.float32),
            jax.ShapeDtypeStruct((N, 1), jnp.float32),
        ],
    )(offsets_t, boxes_r, labels_r, priors_t)

    neg, posce = pl.pallas_call(
        _ce_kernel,
        grid=(N,),
        in_specs=[
            pl.BlockSpec((1, N_PRIORS, N_CLASSES), lambda i: (i, 0, 0)),
            pl.BlockSpec((1, 1, N_PRIORS), lambda i: (i, 0, 0)),
        ],
        out_specs=[
            pl.BlockSpec((1, 1, N_PRIORS), lambda i: (i, 0, 0)),
            pl.BlockSpec((1, 1, 1), lambda i: (i, 0, 0),
                         memory_space=pltpu.SMEM),
        ],
        out_shape=[
            jax.ShapeDtypeStruct((N, 1, N_PRIORS), jnp.float32),
            jax.ShapeDtypeStruct((N, 1, 1), jnp.float32),
        ],
        compiler_params=pltpu.CompilerParams(
            dimension_semantics=("arbitrary",)),
    )(predicted_scores, cls)

    loss = pl.pallas_call(
        _mine_kernel,
        in_specs=[
            pl.BlockSpec((N, 1, N_PRIORS), lambda: (0, 0, 0)),
            pl.BlockSpec((N, 1, 1), lambda: (0, 0, 0)),
            pl.BlockSpec((N, 1), lambda: (0, 0)),
            pl.BlockSpec((N, 1), lambda: (0, 0)),
        ],
        out_specs=pl.BlockSpec((1, 1), lambda: (0, 0),
                               memory_space=pltpu.SMEM),
        out_shape=jax.ShapeDtypeStruct((1, 1), jnp.float32),
    )(neg, posce, loc_p, npos)

    return loss[0, 0]


# speedup vs baseline: 2.4799x; 1.0025x over previous
import jax
import jax.numpy as jnp
from jax.experimental import pallas as pl
from jax.experimental.pallas import tpu as pltpu

N = 32
N_PRIORS = 8732
N_CLASSES = 81
N_OBJ = 12
THRESHOLD = 0.5
NEG_POS_RATIO = 3
ALPHA = 1.0


def _match_kernel(offsets_ref, boxes_ref, labels_ref, priors_ref,
                  cls_ref, loc_ref, npos_ref):
    f32 = jnp.float32
    i32 = jnp.int32
    lane = jax.lax.broadcasted_iota(i32, (1, N_PRIORS), 1)

    pcx = priors_ref[0:1, :]
    pcy = priors_ref[1:2, :]
    pw = priors_ref[2:3, :]
    ph = priors_ref[3:4, :]
    px1 = pcx - pw * 0.5
    py1 = pcy - ph * 0.5
    px2 = pcx + pw * 0.5
    py2 = pcy + ph * 0.5
    parea = pw * ph

    best_iou = jnp.zeros((N, N_PRIORS), f32)
    best_obj = jnp.zeros((N, N_PRIORS), i32)
    forced_obj = jnp.zeros((N, N_PRIORS), i32)
    forced_any = jnp.zeros((N, N_PRIORS), jnp.bool_)

    for j in range(N_OBJ):
        bx1 = boxes_ref[j, 0]
        by1 = boxes_ref[j, 1]
        bx2 = boxes_ref[j, 2]
        by2 = boxes_ref[j, 3]
        barea = (bx2 - bx1) * (by2 - by1)
        iw = jnp.minimum(px2, bx2) - jnp.maximum(px1, bx1)
        ih = jnp.minimum(py2, by2) - jnp.maximum(py1, by1)
        iw = jnp.maximum(iw, 0.0)
        ih = jnp.maximum(ih, 0.0)
        inter = iw * ih
        iou = inter / (parea + barea - inter)
        upd = iou > best_iou
        best_iou = jnp.where(upd, iou, best_iou)
        best_obj = jnp.where(upd, j, best_obj)
        mj = jnp.max(iou, axis=1, keepdims=True)
        oj = jnp.min(jnp.where(iou == mj, lane, N_PRIORS), axis=1,
                     keepdims=True)
        hit = lane == oj
        forced_obj = jnp.where(hit, j, forced_obj)
        forced_any = jnp.logical_or(forced_any, hit)

    obj_eff = jnp.where(forced_any, forced_obj, best_obj)
    iou_eff = jnp.where(forced_any, 1.0, best_iou)

    cls = jnp.zeros((N, N_PRIORS), i32)
    bcx = jnp.zeros((N, N_PRIORS), f32)
    bcy = jnp.zeros((N, N_PRIORS), f32)
    lbw = jnp.zeros((N, N_PRIORS), f32)
    lbh = jnp.zeros((N, N_PRIORS), f32)
    for j in range(N_OBJ):
        sel = obj_eff == j
        cls = jnp.where(sel, labels_ref[j], cls)
        bx1 = boxes_ref[j, 0]
        by1 = boxes_ref[j, 1]
        bx2 = boxes_ref[j, 2]
        by2 = boxes_ref[j, 3]
        bcx = jnp.where(sel, (bx1 + bx2) * 0.5, bcx)
        bcy = jnp.where(sel, (by1 + by2) * 0.5, bcy)
        lbw = jnp.where(sel, jnp.log(bx2 - bx1), lbw)
        lbh = jnp.where(sel, jnp.log(by2 - by1), lbh)

    cls = jnp.where(iou_eff < THRESHOLD, 0, cls)
    pos_f = (cls != 0).astype(f32)
    npos_ref[:, :] = jnp.sum(pos_f, axis=1, keepdims=True)

    inv_pw = 10.0 / pw
    inv_ph = 10.0 / ph
    t0 = (bcx - pcx) * inv_pw
    t1 = (bcy - pcy) * inv_ph
    t2 = (lbw - jnp.log(pw)) * 5.0
    t3 = (lbh - jnp.log(ph)) * 5.0
    pos_half = pos_f * 0.5
    loc = jnp.zeros((N, 1), f32)
    for k, t in enumerate((t0, t1, t2, t3)):
        d = offsets_ref[:, k, :] - t
        ad = jnp.abs(d)
        m = jnp.minimum(ad, 1.0)
        loc = loc + jnp.sum(m * (ad + ad - m) * pos_half, axis=1,
                            keepdims=True)
    loc_ref[:, :] = loc
    cls_ref[:, 0, :] = cls


def _ce_kernel(scores_ref, cls_ref, neg_ref, posce_ref):
    i32 = jnp.int32
    st = scores_ref[0].T
    e = jnp.exp(st)
    lse = jnp.log(jnp.sum(e, axis=0, keepdims=True))
    cls_row = cls_ref[0, 0, :].reshape(1, N_PRIORS)
    rio = jax.lax.broadcasted_iota(i32, (N_CLASSES, N_PRIORS), 0)
    selsc = jnp.sum(jnp.where(rio == cls_row, st, 0.0), axis=0, keepdims=True)
    ce_row = lse - selsc
    pos = cls_row != 0
    posce_ref[0, 0, 0] = jnp.sum(jnp.where(pos, ce_row, 0.0))
    neg_ref[0, 0, :] = jnp.where(pos, 0.0, ce_row)[0]


def _mine_kernel(neg_ref, posce_ref, loc_partial_ref, npos_ref, out_ref):
    f32 = jnp.float32
    i32 = jnp.int32
    neg = neg_ref[:, 0, :]
    nbits = jax.lax.bitcast_convert_type(neg, i32)
    k_hard = (NEG_POS_RATIO * npos_ref[:, :]).astype(i32)

    def body(_, carry):
        lo, hi = carry
        mid = lo + (hi - lo + 1) // 2
        cnt = jnp.sum((nbits >= mid).astype(i32), axis=1, keepdims=True)
        ok = cnt >= k_hard
        return jnp.where(ok, mid, lo), jnp.where(ok, hi, mid - 1)

    lo, _ = jax.lax.fori_loop(
        0, 31, body,
        (jnp.zeros((N, 1), i32), jnp.full((N, 1), 0x7F7FFFFF, i32)))
    kth = jnp.max(jnp.where(nbits == lo, neg, 0.0), axis=1, keepdims=True)
    gt = neg > kth
    cnt_gt = jnp.sum(gt.astype(i32), axis=1, keepdims=True)
    hard = jnp.sum(jnp.where(gt, neg, 0.0), axis=1, keepdims=True) + (
        (k_hard - cnt_gt).astype(f32) * kth)

    loc_sum = jnp.sum(loc_partial_ref[:, :])
    n_pos = jnp.sum(npos_ref[:, :])
    conf = jnp.sum(posce_ref[:, 0, :]) + jnp.sum(hard)
    out_ref[0, 0] = ALPHA * loc_sum / (n_pos * 4.0) + conf / n_pos


@jax.jit
def kernel(predicted_offsets, predicted_scores, boxes, labels, priors_cxcy):
    offsets_t = jnp.transpose(predicted_offsets, (0, 2, 1))
    priors_t = priors_cxcy.T
    boxes_r = jnp.transpose(boxes, (1, 2, 0))[:, :, :, None]
    labels_r = labels.astype(jnp.int32).T[:, :, None]

    cls, loc_p, npos = pl.pallas_call(
        _match_kernel,
        in_specs=[
            pl.BlockSpec((N, 4, N_PRIORS), lambda: (0, 0, 0)),
            pl.BlockSpec((N_OBJ, 4, N, 1), lambda: (0, 0, 0, 0)),
            pl.BlockSpec((N_OBJ, N, 1), lambda: (0, 0, 0)),
            pl.BlockSpec((4, N_PRIORS), lambda: (0, 0)),
        ],
        out_specs=[
            pl.BlockSpec((N, 1, N_PRIORS), lambda: (0, 0, 0)),
            pl.BlockSpec((N, 1), lambda: (0, 0)),
            pl.BlockSpec((N, 1), lambda: (0, 0)),
        ],
        out_shape=[
            jax.ShapeDtypeStruct((N, 1, N_PRIORS), jnp.int32),
            jax.ShapeDtypeStruct((N, 1), jnp.float32),
            jax.ShapeDtypeStruct((N, 1), jnp.float32),
        ],
    )(offsets_t, boxes_r, labels_r, priors_t)

    neg, posce = pl.pallas_call(
        _ce_kernel,
        grid=(N,),
        in_specs=[
            pl.BlockSpec((1, N_PRIORS, N_CLASSES), lambda i: (i, 0, 0)),
            pl.BlockSpec((1, 1, N_PRIORS), lambda i: (i, 0, 0)),
        ],
        out_specs=[
            pl.BlockSpec((1, 1, N_PRIORS), lambda i: (i, 0, 0)),
            pl.BlockSpec((1, 1, 1), lambda i: (i, 0, 0),
                         memory_space=pltpu.SMEM),
        ],
        out_shape=[
            jax.ShapeDtypeStruct((N, 1, N_PRIORS), jnp.float32),
            jax.ShapeDtypeStruct((N, 1, 1), jnp.float32),
        ],
        compiler_params=pltpu.CompilerParams(
            dimension_semantics=("arbitrary",)),
    )(predicted_scores, cls)

    loss = pl.pallas_call(
        _mine_kernel,
        in_specs=[
            pl.BlockSpec((N, 1, N_PRIORS), lambda: (0, 0, 0)),
            pl.BlockSpec((N, 1, 1), lambda: (0, 0, 0)),
            pl.BlockSpec((N, 1), lambda: (0, 0)),
            pl.BlockSpec((N, 1), lambda: (0, 0)),
        ],
        out_specs=pl.BlockSpec((1, 1), lambda: (0, 0),
                               memory_space=pltpu.SMEM),
        out_shape=jax.ShapeDtypeStruct((1, 1), jnp.float32),
    )(neg, posce, loc_p, npos)

    return loss[0, 0]


# speedup vs baseline: 2.5303x; 1.0203x over previous
import jax
import jax.numpy as jnp
from jax.experimental import pallas as pl
from jax.experimental.pallas import tpu as pltpu

N = 32
N_PRIORS = 8732
N_CLASSES = 81
N_OBJ = 12
THRESHOLD = 0.5
NEG_POS_RATIO = 3
ALPHA = 1.0


def _match_kernel(offsets_ref, boxes_ref, labels_ref, priors_ref,
                  cls_ref, loc_ref, npos_ref):
    f32 = jnp.float32
    i32 = jnp.int32
    lane = jax.lax.broadcasted_iota(i32, (1, N_PRIORS), 1)

    pcx = priors_ref[0:1, :]
    pcy = priors_ref[1:2, :]
    pw = priors_ref[2:3, :]
    ph = priors_ref[3:4, :]
    px1 = pcx - pw * 0.5
    py1 = pcy - ph * 0.5
    px2 = pcx + pw * 0.5
    py2 = pcy + ph * 0.5
    parea = pw * ph

    best_iou = jnp.zeros((N, N_PRIORS), f32)
    best_obj = jnp.zeros((N, N_PRIORS), i32)
    forced_obj = jnp.zeros((N, N_PRIORS), i32)
    forced_any = jnp.zeros((N, N_PRIORS), jnp.bool_)

    for j in range(N_OBJ):
        bx1 = boxes_ref[j, 0]
        by1 = boxes_ref[j, 1]
        bx2 = boxes_ref[j, 2]
        by2 = boxes_ref[j, 3]
        barea = (bx2 - bx1) * (by2 - by1)
        iw = jnp.minimum(px2, bx2) - jnp.maximum(px1, bx1)
        ih = jnp.minimum(py2, by2) - jnp.maximum(py1, by1)
        iw = jnp.maximum(iw, 0.0)
        ih = jnp.maximum(ih, 0.0)
        inter = iw * ih
        iou = inter / (parea + barea - inter)
        upd = iou > best_iou
        best_iou = jnp.where(upd, iou, best_iou)
        best_obj = jnp.where(upd, j, best_obj)
        mj = jnp.max(iou, axis=1, keepdims=True)
        oj = jnp.min(jnp.where(iou == mj, lane, N_PRIORS), axis=1,
                     keepdims=True)
        hit = lane == oj
        forced_obj = jnp.where(hit, j, forced_obj)
        forced_any = jnp.logical_or(forced_any, hit)

    obj_eff = jnp.where(forced_any, forced_obj, best_obj)
    iou_eff = jnp.where(forced_any, 1.0, best_iou)

    cls = jnp.zeros((N, N_PRIORS), i32)
    bcx = jnp.zeros((N, N_PRIORS), f32)
    bcy = jnp.zeros((N, N_PRIORS), f32)
    lbw = jnp.zeros((N, N_PRIORS), f32)
    lbh = jnp.zeros((N, N_PRIORS), f32)
    for j in range(N_OBJ):
        sel = obj_eff == j
        cls = jnp.where(sel, labels_ref[j], cls)
        bx1 = boxes_ref[j, 0]
        by1 = boxes_ref[j, 1]
        bx2 = boxes_ref[j, 2]
        by2 = boxes_ref[j, 3]
        bcx = jnp.where(sel, (bx1 + bx2) * 0.5, bcx)
        bcy = jnp.where(sel, (by1 + by2) * 0.5, bcy)
        lbw = jnp.where(sel, jnp.log(bx2 - bx1), lbw)
        lbh = jnp.where(sel, jnp.log(by2 - by1), lbh)

    cls = jnp.where(iou_eff < THRESHOLD, 0, cls)
    pos_f = (cls != 0).astype(f32)
    npos_ref[:, :] = jnp.sum(pos_f, axis=1, keepdims=True)

    t0 = (bcx - pcx) * 10.0 / pw
    t1 = (bcy - pcy) * 10.0 / ph
    t2 = (lbw - jnp.log(pw)) * 5.0
    t3 = (lbh - jnp.log(ph)) * 5.0
    loc = jnp.zeros((N, 1), f32)
    for k, t in enumerate((t0, t1, t2, t3)):
        d = offsets_ref[:, k, :] - t
        ad = jnp.abs(d)
        sl1 = jnp.where(ad < 1.0, 0.5 * d * d, ad - 0.5)
        loc = loc + jnp.sum(sl1 * pos_f, axis=1, keepdims=True)
    loc_ref[:, :] = loc
    cls_ref[:, 0, :] = cls


CE_B = 4


def _ce_kernel(scores_ref, cls_ref, neg_ref, posce_ref):
    i32 = jnp.int32
    rio = jax.lax.broadcasted_iota(i32, (N_CLASSES, N_PRIORS), 0)
    for b in range(CE_B):
        st = scores_ref[b].T
        e = jnp.exp(st)
        lse = jnp.log(jnp.sum(e, axis=0, keepdims=True))
        cls_row = cls_ref[b, 0, :].reshape(1, N_PRIORS)
        selsc = jnp.sum(jnp.where(rio == cls_row, st, 0.0), axis=0,
                        keepdims=True)
        ce_row = lse - selsc
        pos = cls_row != 0
        posce_ref[b, 0, 0] = jnp.sum(jnp.where(pos, ce_row, 0.0))
        neg_ref[b, 0, :] = jnp.where(pos, 0.0, ce_row)[0]


def _mine_kernel(neg_ref, posce_ref, loc_partial_ref, npos_ref, out_ref):
    f32 = jnp.float32
    i32 = jnp.int32
    neg = neg_ref[:, 0, :]
    nbits = jax.lax.bitcast_convert_type(neg, i32)
    k_hard = (NEG_POS_RATIO * npos_ref[:, :]).astype(i32)

    def body(_, carry):
        lo, hi = carry
        mid = lo + (hi - lo + 1) // 2
        cnt = jnp.sum((nbits >= mid).astype(i32), axis=1, keepdims=True)
        ok = cnt >= k_hard
        return jnp.where(ok, mid, lo), jnp.where(ok, hi, mid - 1)

    lo, _ = jax.lax.fori_loop(
        0, 31, body,
        (jnp.zeros((N, 1), i32), jnp.full((N, 1), 0x7F7FFFFF, i32)))
    kth = jnp.max(jnp.where(nbits == lo, neg, 0.0), axis=1, keepdims=True)
    gt = neg > kth
    cnt_gt = jnp.sum(gt.astype(i32), axis=1, keepdims=True)
    hard = jnp.sum(jnp.where(gt, neg, 0.0), axis=1, keepdims=True) + (
        (k_hard - cnt_gt).astype(f32) * kth)

    loc_sum = jnp.sum(loc_partial_ref[:, :])
    n_pos = jnp.sum(npos_ref[:, :])
    conf = jnp.sum(posce_ref[:, 0, :]) + jnp.sum(hard)
    out_ref[0, 0] = ALPHA * loc_sum / (n_pos * 4.0) + conf / n_pos


@jax.jit
def kernel(predicted_offsets, predicted_scores, boxes, labels, priors_cxcy):
    offsets_t = jnp.transpose(predicted_offsets, (0, 2, 1))
    priors_t = priors_cxcy.T
    boxes_r = jnp.transpose(boxes, (1, 2, 0))[:, :, :, None]
    labels_r = labels.astype(jnp.int32).T[:, :, None]

    cls, loc_p, npos = pl.pallas_call(
        _match_kernel,
        in_specs=[
            pl.BlockSpec((N, 4, N_PRIORS), lambda: (0, 0, 0)),
            pl.BlockSpec((N_OBJ, 4, N, 1), lambda: (0, 0, 0, 0)),
            pl.BlockSpec((N_OBJ, N, 1), lambda: (0, 0, 0)),
            pl.BlockSpec((4, N_PRIORS), lambda: (0, 0)),
        ],
        out_specs=[
            pl.BlockSpec((N, 1, N_PRIORS), lambda: (0, 0, 0)),
            pl.BlockSpec((N, 1), lambda: (0, 0)),
            pl.BlockSpec((N, 1), lambda: (0, 0)),
        ],
        out_shape=[
            jax.ShapeDtypeStruct((N, 1, N_PRIORS), jnp.int32),
            jax.ShapeDtypeStruct((N, 1), jnp.float32),
            jax.ShapeDtypeStruct((N, 1), jnp.float32),
        ],
    )(offsets_t, boxes_r, labels_r, priors_t)

    neg, posce = pl.pallas_call(
        _ce_kernel,
        grid=(N // CE_B,),
        in_specs=[
            pl.BlockSpec((CE_B, N_PRIORS, N_CLASSES), lambda i: (i, 0, 0)),
            pl.BlockSpec((CE_B, 1, N_PRIORS), lambda i: (i, 0, 0)),
        ],
        out_specs=[
            pl.BlockSpec((CE_B, 1, N_PRIORS), lambda i: (i, 0, 0)),
            pl.BlockSpec((CE_B, 1, 1), lambda i: (i, 0, 0),
                         memory_space=pltpu.SMEM),
        ],
        out_shape=[
            jax.ShapeDtypeStruct((N, 1, N_PRIORS), jnp.float32),
            jax.ShapeDtypeStruct((N, 1, 1), jnp.float32),
        ],
        compiler_params=pltpu.CompilerParams(
            dimension_semantics=("arbitrary",)),
    )(predicted_scores, cls)

    loss = pl.pallas_call(
        _mine_kernel,
        in_specs=[
            pl.BlockSpec((N, 1, N_PRIORS), lambda: (0, 0, 0)),
            pl.BlockSpec((N, 1, 1), lambda: (0, 0, 0)),
            pl.BlockSpec((N, 1), lambda: (0, 0)),
            pl.BlockSpec((N, 1), lambda: (0, 0)),
        ],
        out_specs=pl.BlockSpec((1, 1), lambda: (0, 0),
                               memory_space=pltpu.SMEM),
        out_shape=jax.ShapeDtypeStruct((1, 1), jnp.float32),
    )(neg, posce, loc_p, npos)

    return loss[0, 0]


# speedup vs baseline: 2.6282x; 1.0387x over previous
import jax
import jax.numpy as jnp
from jax.experimental import pallas as pl
from jax.experimental.pallas import tpu as pltpu

N = 32
N_PRIORS = 8732
N_CLASSES = 81
N_OBJ = 12
THRESHOLD = 0.5
NEG_POS_RATIO = 3
ALPHA = 1.0
CE_B = 4


def _match_kernel(offsets_ref, boxes_ref, labels_ref, priors_ref,
                  cls_ref, loc_ref, npos_ref):
    f32 = jnp.float32
    i32 = jnp.int32
    lane = jax.lax.broadcasted_iota(i32, (1, N_PRIORS), 1)

    pcx = priors_ref[0:1, :]
    pcy = priors_ref[1:2, :]
    pw = priors_ref[2:3, :]
    ph = priors_ref[3:4, :]
    px1 = pcx - pw * 0.5
    py1 = pcy - ph * 0.5
    px2 = pcx + pw * 0.5
    py2 = pcy + ph * 0.5
    parea = pw * ph

    best_iou = jnp.zeros((N, N_PRIORS), f32)
    best_obj = jnp.zeros((N, N_PRIORS), i32)
    forced_obj = jnp.zeros((N, N_PRIORS), i32)
    forced_any = jnp.zeros((N, N_PRIORS), jnp.bool_)

    for j in range(N_OBJ):
        bx1 = boxes_ref[j, 0]
        by1 = boxes_ref[j, 1]
        bx2 = boxes_ref[j, 2]
        by2 = boxes_ref[j, 3]
        barea = (bx2 - bx1) * (by2 - by1)
        iw = jnp.minimum(px2, bx2) - jnp.maximum(px1, bx1)
        ih = jnp.minimum(py2, by2) - jnp.maximum(py1, by1)
        iw = jnp.maximum(iw, 0.0)
        ih = jnp.maximum(ih, 0.0)
        inter = iw * ih
        iou = inter / (parea + barea - inter)
        upd = iou > best_iou
        best_iou = jnp.where(upd, iou, best_iou)
        best_obj = jnp.where(upd, j, best_obj)
        mj = jnp.max(iou, axis=1, keepdims=True)
        oj = jnp.min(jnp.where(iou == mj, lane, N_PRIORS), axis=1,
                     keepdims=True)
        hit = lane == oj
        forced_obj = jnp.where(hit, j, forced_obj)
        forced_any = jnp.logical_or(forced_any, hit)

    obj_eff = jnp.where(forced_any, forced_obj, best_obj)
    iou_eff = jnp.where(forced_any, 1.0, best_iou)

    cls = jnp.zeros((N, N_PRIORS), i32)
    bcx = jnp.zeros((N, N_PRIORS), f32)
    bcy = jnp.zeros((N, N_PRIORS), f32)
    lbw = jnp.zeros((N, N_PRIORS), f32)
    lbh = jnp.zeros((N, N_PRIORS), f32)
    for j in range(N_OBJ):
        sel = obj_eff == j
        cls = jnp.where(sel, labels_ref[j], cls)
        bx1 = boxes_ref[j, 0]
        by1 = boxes_ref[j, 1]
        bx2 = boxes_ref[j, 2]
        by2 = boxes_ref[j, 3]
        bcx = jnp.where(sel, (bx1 + bx2) * 0.5, bcx)
        bcy = jnp.where(sel, (by1 + by2) * 0.5, bcy)
        lbw = jnp.where(sel, jnp.log(bx2 - bx1), lbw)
        lbh = jnp.where(sel, jnp.log(by2 - by1), lbh)

    cls = jnp.where(iou_eff < THRESHOLD, 0, cls)
    pos_f = (cls != 0).astype(f32)
    npos_ref[:, :] = jnp.sum(pos_f, axis=1, keepdims=True)

    inv_pw = 10.0 / pw
    inv_ph = 10.0 / ph
    t0 = (bcx - pcx) * inv_pw
    t1 = (bcy - pcy) * inv_ph
    t2 = (lbw - jnp.log(pw)) * 5.0
    t3 = (lbh - jnp.log(ph)) * 5.0
    pos_half = pos_f * 0.5
    loc = jnp.zeros((N, 1), f32)
    for k, t in enumerate((t0, t1, t2, t3)):
        d = offsets_ref[:, k, :] - t
        ad = jnp.abs(d)
        m = jnp.minimum(ad, 1.0)
        loc = loc + jnp.sum(m * (ad + ad - m) * pos_half, axis=1,
                            keepdims=True)
    loc_ref[:, :] = loc
    cls_ref[:, 0, :] = cls


def _ce_mine_kernel(scores_ref, cls_ref, npos_ref, posce_ref, hard_ref):
    f32 = jnp.float32
    i32 = jnp.int32
    rio = jax.lax.broadcasted_iota(i32, (N_CLASSES, N_PRIORS), 0)
    negs = []
    poss = []
    for b in range(CE_B):
        st = scores_ref[b].T
        e = jnp.exp(st)
        lse = jnp.log(jnp.sum(e, axis=0, keepdims=True))
        cls_row = cls_ref[b, 0, :].reshape(1, N_PRIORS)
        selsc = jnp.sum(jnp.where(rio == cls_row, st, 0.0), axis=0,
                        keepdims=True)
        ce_row = lse - selsc
        pos = cls_row != 0
        poss.append(jnp.where(pos, ce_row, 0.0))
        negs.append(jnp.where(pos, 0.0, ce_row))

    posce_ref[0] = jnp.sum(jnp.concatenate(poss, axis=0), axis=1,
                           keepdims=True)
    neg = jnp.concatenate(negs, axis=0)
    nbits = jax.lax.bitcast_convert_type(neg, i32)
    k_hard = (NEG_POS_RATIO * npos_ref[0]).astype(i32)

    def body(_, carry):
        lo, hi = carry
        mid = lo + (hi - lo + 1) // 2
        cnt = jnp.sum((nbits >= mid).astype(i32), axis=1, keepdims=True)
        ok = cnt >= k_hard
        return jnp.where(ok, mid, lo), jnp.where(ok, hi, mid - 1)

    lo, _ = jax.lax.fori_loop(
        0, 31, body,
        (jnp.zeros((CE_B, 1), i32), jnp.full((CE_B, 1), 0x7F7FFFFF, i32)))
    kth = jnp.max(jnp.where(nbits == lo, neg, 0.0), axis=1, keepdims=True)
    gt = neg > kth
    cnt_gt = jnp.sum(gt.astype(i32), axis=1, keepdims=True)
    hard_ref[0] = jnp.sum(jnp.where(gt, neg, 0.0), axis=1, keepdims=True) + (
        (k_hard - cnt_gt).astype(f32) * kth)


def _finish_kernel(loc_partial_ref, npos_ref, posce_ref, hard_ref, out_ref):
    loc_sum = jnp.sum(loc_partial_ref[:, :])
    n_pos = jnp.sum(npos_ref[:, :])
    conf = jnp.sum(posce_ref[:, :, :]) + jnp.sum(hard_ref[:, :, :])
    out_ref[0, 0] = ALPHA * loc_sum / (n_pos * 4.0) + conf / n_pos


@jax.jit
def kernel(predicted_offsets, predicted_scores, boxes, labels, priors_cxcy):
    offsets_t = jnp.transpose(predicted_offsets, (0, 2, 1))
    priors_t = priors_cxcy.T
    boxes_r = jnp.transpose(boxes, (1, 2, 0))[:, :, :, None]
    labels_r = labels.astype(jnp.int32).T[:, :, None]

    cls, loc_p, npos = pl.pallas_call(
        _match_kernel,
        in_specs=[
            pl.BlockSpec((N, 4, N_PRIORS), lambda: (0, 0, 0)),
            pl.BlockSpec((N_OBJ, 4, N, 1), lambda: (0, 0, 0, 0)),
            pl.BlockSpec((N_OBJ, N, 1), lambda: (0, 0, 0)),
            pl.BlockSpec((4, N_PRIORS), lambda: (0, 0)),
        ],
        out_specs=[
            pl.BlockSpec((N, 1, N_PRIORS), lambda: (0, 0, 0)),
            pl.BlockSpec((N, 1), lambda: (0, 0)),
            pl.BlockSpec((N, 1), lambda: (0, 0)),
        ],
        out_shape=[
            jax.ShapeDtypeStruct((N, 1, N_PRIORS), jnp.int32),
            jax.ShapeDtypeStruct((N, 1), jnp.float32),
            jax.ShapeDtypeStruct((N, 1), jnp.float32),
        ],
    )(offsets_t, boxes_r, labels_r, priors_t)

    npos_r = npos.reshape(N // CE_B, CE_B, 1)

    posce, hard = pl.pallas_call(
        _ce_mine_kernel,
        grid=(N // CE_B,),
        in_specs=[
            pl.BlockSpec((CE_B, N_PRIORS, N_CLASSES), lambda i: (i, 0, 0)),
            pl.BlockSpec((CE_B, 1, N_PRIORS), lambda i: (i, 0, 0)),
            pl.BlockSpec((1, CE_B, 1), lambda i: (i, 0, 0)),
        ],
        out_specs=[
            pl.BlockSpec((1, CE_B, 1), lambda i: (i, 0, 0)),
            pl.BlockSpec((1, CE_B, 1), lambda i: (i, 0, 0)),
        ],
        out_shape=[
            jax.ShapeDtypeStruct((N // CE_B, CE_B, 1), jnp.float32),
            jax.ShapeDtypeStruct((N // CE_B, CE_B, 1), jnp.float32),
        ],
        compiler_params=pltpu.CompilerParams(
            dimension_semantics=("arbitrary",)),
    )(predicted_scores, cls, npos_r)

    loss = pl.pallas_call(
        _finish_kernel,
        in_specs=[
            pl.BlockSpec((N, 1), lambda: (0, 0)),
            pl.BlockSpec((N, 1), lambda: (0, 0)),
            pl.BlockSpec((N // CE_B, CE_B, 1), lambda: (0, 0, 0)),
            pl.BlockSpec((N // CE_B, CE_B, 1), lambda: (0, 0, 0)),
        ],
        out_specs=pl.BlockSpec((1, 1), lambda: (0, 0),
                               memory_space=pltpu.SMEM),
        out_shape=jax.ShapeDtypeStruct((1, 1), jnp.float32),
    )(loc_p, npos, posce, hard)

    return loss[0, 0]
